# Initial kernel scaffold; baseline (speedup 1.0000x reference)
import jax
import jax.numpy as jnp
from jax.experimental import pallas as pl


def kernel(h_src_user, h_src_item, h_dst_user, h_dst_item, edge_u2i, edge_i2u, fc_W, attn_l, attn_r, W_w, W_b):
    raise NotImplementedError("write your pallas kernel here")


# speedup vs baseline: 20.1847x; 20.1847x over previous
import functools

import jax
import jax.numpy as jnp
from jax import lax
from jax.experimental import pallas as pl
from jax.experimental.pallas import tpu as pltpu
from jax.experimental.pallas import tpu_sc as plsc

N = 10000
E = 160000
D_IN = 128
H = 4
D = 128
NP = 10240

NC = 2
NS = 16
EPT = E // NS
C = 80
NCHUNK = EPT // C
STRIPE = NP // NS


def _proj_body(hsu, hsi, hdu, hdi, fw, P, zu, zi, psu, psi, pdu, pdi):
    zu[...] = jnp.dot(hsu[...], fw[...], preferred_element_type=jnp.float32)
    zi[...] = jnp.dot(hsi[...], fw[...], preferred_element_type=jnp.float32)
    psu[...] = jnp.dot(hsu[...], P[...], preferred_element_type=jnp.float32)
    psi[...] = jnp.dot(hsi[...], P[...], preferred_element_type=jnp.float32)
    pdu[...] = jnp.dot(hdu[...], P[...], preferred_element_type=jnp.float32)
    pdi[...] = jnp.dot(hdi[...], P[...], preferred_element_type=jnp.float32)


def _projections(hsu, hsi, hdu, hdi, fc_W, P):
    RB = 1000
    grid = N // RB
    feat_spec = pl.BlockSpec((RB, D_IN), lambda i: (i, 0))
    out8_spec = pl.BlockSpec((RB, 8), lambda i: (i, 0))
    return pl.pallas_call(
        _proj_body,
        grid=(grid,),
        in_specs=[feat_spec, feat_spec, feat_spec, feat_spec,
                  pl.BlockSpec((D_IN, H * D), lambda i: (0, 0)),
                  pl.BlockSpec((D_IN, 8), lambda i: (0, 0))],
        out_specs=[pl.BlockSpec((RB, H * D), lambda i: (i, 0)),
                   pl.BlockSpec((RB, H * D), lambda i: (i, 0)),
                   out8_spec, out8_spec, out8_spec, out8_spec],
        out_shape=[jax.ShapeDtypeStruct((N, H * D), jnp.float32),
                   jax.ShapeDtypeStruct((N, H * D), jnp.float32)]
                  + [jax.ShapeDtypeStruct((N, 8), jnp.float32)] * 4,
    )(hsu, hsi, hdu, hdi, fc_W, P)


def _sc_body(zu, zi, el_u, el_i, er_u, er_i, src_u2i, dst_u2i, src_i2u,
             dst_i2u, zrows0, zden0,
             num_i, num_u, den_i, den_u,
             el_t, er_t, src_c, dst_c, sidx_c, ee_c, zrow, mbuf, num_s, den_s,
             sem):
    c = lax.axis_index("c")
    s = lax.axis_index("s")
    neg = jnp.full((16,), -3.4e38, jnp.float32)

    def vmax_table(tab, mbuf):
        def body(i, m):
            return jnp.maximum(m, tab[pl.ds(i * 16, 16)])
        m = lax.fori_loop(0, N // 16, body, neg)
        acc = m[0]
        for i in range(1, 16):
            acc = jnp.maximum(acc, m[i])
        return acc

    for t in range(4):
        rel = t % 2
        h_loc = t // 2
        h = c * 2 + h_loc
        if rel == 0:
            z, el_x, er_x = zu, el_u, er_i
            e_src, e_dst = src_u2i, dst_u2i
            num_o, den_o = num_i, den_i
        else:
            z, el_x, er_x = zi, el_i, er_u
            e_src, e_dst = src_i2u, dst_i2u
            num_o, den_o = num_u, den_u

        pltpu.sync_copy(el_x.at[h], el_t)
        pltpu.sync_copy(er_x.at[h], er_t)
        M = jnp.maximum(vmax_table(el_t, mbuf) + vmax_table(er_t, mbuf), 0.0)

        for k in range(5):
            pltpu.sync_copy(zrows0, num_s.at[pl.ds(s * STRIPE + k * 128, 128)])
        pltpu.sync_copy(zden0, den_s.at[pl.ds(s * STRIPE, STRIPE)])
        plsc.subcore_barrier()

        def chunk_body(i, carry):
            base = s * EPT + i * C
            pltpu.sync_copy(e_src.at[pl.ds(base, C)], src_c)
            pltpu.sync_copy(e_dst.at[pl.ds(base, C)], dst_c)
            for j in range(C // 16):
                sv = src_c[pl.ds(j * 16, 16)]
                dv = dst_c[pl.ds(j * 16, 16)]
                x = plsc.load_gather(el_t, [sv]) + plsc.load_gather(er_t, [dv])
                e = jnp.where(x > 0, x, 0.2 * x)
                ee_c[pl.ds(j * 16, 16)] = jnp.exp(e - M)
                sidx_c[pl.ds(j * 16, 16)] = sv * 4 + h
            pltpu.async_copy(z.at[sidx_c], zrow, sem).wait()

            def row_body(r, carry2):
                spl = plsc.load_gather(ee_c, [jnp.full((16,), r, jnp.int32)])
                for k in range(8):
                    zrow[r, pl.ds(k * 16, 16)] = zrow[r, pl.ds(k * 16, 16)] * spl
                return carry2
            lax.fori_loop(0, C, row_body, 0)

            pltpu.sync_copy(zrow, num_s.at[dst_c], add=True)
            pltpu.sync_copy(ee_c, den_s.at[dst_c], add=True)
            return carry
        lax.fori_loop(0, NCHUNK, chunk_body, 0)
        plsc.subcore_barrier()

        pltpu.sync_copy(num_s.at[pl.ds(s * STRIPE, STRIPE)],
                        num_o.at[h, pl.ds(s * STRIPE, STRIPE)])
        pltpu.sync_copy(den_s.at[pl.ds(s * STRIPE, STRIPE)],
                        den_o.at[h, pl.ds(s * STRIPE, STRIPE)])
        plsc.subcore_barrier()


def _sc_aggregate(zu_flat, zi_flat, el_u, el_i, er_u, er_i, e_u2i, e_i2u):
    src_u2i, dst_u2i = e_u2i[0], e_u2i[1]
    src_i2u, dst_i2u = e_i2u[0], e_i2u[1]
    zrows0 = jnp.zeros((128, D), jnp.float32)
    zden0 = jnp.zeros((STRIPE,), jnp.float32)
    mesh = plsc.VectorSubcoreMesh(core_axis_name="c", subcore_axis_name="s")
    f = pl.kernel(
        _sc_body,
        out_type=[jax.ShapeDtypeStruct((H, NP, D), jnp.float32),
                  jax.ShapeDtypeStruct((H, NP, D), jnp.float32),
                  jax.ShapeDtypeStruct((H, NP), jnp.float32),
                  jax.ShapeDtypeStruct((H, NP), jnp.float32)],
        mesh=mesh,
        compiler_params=pltpu.CompilerParams(needs_layout_passes=False),
        scratch_types=[
            pltpu.VMEM((N,), jnp.float32),
            pltpu.VMEM((N,), jnp.float32),
            pltpu.VMEM((C,), jnp.int32),
            pltpu.VMEM((C,), jnp.int32),
            pltpu.VMEM((C,), jnp.int32),
            pltpu.VMEM((C,), jnp.float32),
            pltpu.VMEM((C, D), jnp.float32),
            pltpu.VMEM((16,), jnp.float32),
            pltpu.VMEM_SHARED((NP, D), jnp.float32),
            pltpu.VMEM_SHARED((NP,), jnp.float32),
            pltpu.SemaphoreType.DMA,
        ],
    )
    return f(zu_flat, zi_flat, el_u, el_i, er_u, er_i,
             src_u2i, dst_u2i, src_i2u, dst_i2u, zrows0, zden0)


def _fin_body(num_u, den_u, hdu, num_i, den_i, hdi, W1, W2, b, zu, zi):
    def one(num_ref, den_ref, feat_ref, out_ref):
        den = jnp.maximum(den_ref[...], 1e-9)
        acc = num_ref[0] / den[:, 0:1]
        for h in range(1, H):
            acc = acc + num_ref[h] / den[:, h:h + 1]
        nu = acc * (1.0 / H)
        y = (jnp.dot(nu, W1[...], preferred_element_type=jnp.float32)
             + jnp.dot(feat_ref[...], W2[...], preferred_element_type=jnp.float32)
             + b[...])
        y = jnp.maximum(y, 0.0)
        nrm = jnp.sqrt(jnp.sum(y * y, axis=1, keepdims=True))
        out_ref[...] = y / jnp.where(nrm == 0.0, 1.0, nrm)
    one(num_u, den_u, hdu, zu)
    one(num_i, den_i, hdi, zi)


def _finalize(num_u, den_uT, hdu, num_i, den_iT, hdi, W1, W2, b):
    RB = 1000
    grid = N // RB
    num_spec = pl.BlockSpec((H, RB, D), lambda i: (0, i, 0))
    den_spec = pl.BlockSpec((RB, H), lambda i: (i, 0))
    feat_spec = pl.BlockSpec((RB, D_IN), lambda i: (i, 0))
    w_spec = pl.BlockSpec((128, D), lambda i: (0, 0))
    return pl.pallas_call(
        _fin_body,
        grid=(grid,),
        in_specs=[num_spec, den_spec, feat_spec,
                  num_spec, den_spec, feat_spec,
                  w_spec, w_spec, pl.BlockSpec((1, D), lambda i: (0, 0))],
        out_specs=[pl.BlockSpec((RB, D), lambda i: (i, 0)),
                   pl.BlockSpec((RB, D), lambda i: (i, 0))],
        out_shape=[jax.ShapeDtypeStruct((N, D), jnp.float32),
                   jax.ShapeDtypeStruct((N, D), jnp.float32)],
    )(num_u, den_uT, hdu, num_i, den_iT, hdi, W1, W2, b)


def kernel(h_src_user, h_src_item, h_dst_user, h_dst_item, edge_u2i, edge_i2u,
           fc_W, attn_l, attn_r, W_w, W_b):
    fc_r = fc_W.reshape(D_IN, H, D)
    Wl = jnp.einsum('khd,hd->kh', fc_r, attn_l)
    Wr = jnp.einsum('khd,hd->kh', fc_r, attn_r)
    P = jnp.concatenate([Wl, Wr], axis=1)

    zu, zi, psu, psi, pdu, pdi = _projections(
        h_src_user, h_src_item, h_dst_user, h_dst_item, fc_W, P)

    el_u = psu[:, 0:4].T
    el_i = psi[:, 0:4].T
    er_u = pdu[:, 4:8].T
    er_i = pdi[:, 4:8].T

    num_i, num_u, den_i, den_u = _sc_aggregate(
        zu.reshape(N * H, D), zi.reshape(N * H, D),
        el_u, el_i, er_u, er_i, edge_u2i, edge_i2u)

    W1 = W_w[:D, :]
    W2 = W_w[D:, :]
    z_user, z_item = _finalize(
        num_u, den_u[:, :N].T, h_dst_user,
        num_i, den_i[:, :N].T, h_dst_item,
        W1, W2, W_b.reshape(1, D))
    return (z_user, z_item)


# speedup vs baseline: 34.3021x; 1.6994x over previous
import functools

import jax
import jax.numpy as jnp
from jax import lax
from jax.experimental import pallas as pl
from jax.experimental.pallas import tpu as pltpu
from jax.experimental.pallas import tpu_sc as plsc

N = 10000
E = 160000
D_IN = 128
H = 4
D = 128
NP = 10240

NC = 2
NS = 16
EPT = E // NS
C = 80
NCHUNK = EPT // C
STRIPE = NP // NS


def _proj_body(hsu, hsi, hdu, hdi, fw, P, zu, zi, psu, psi, pdu, pdi):
    zu[...] = jnp.dot(hsu[...], fw[...], preferred_element_type=jnp.float32)
    zi[...] = jnp.dot(hsi[...], fw[...], preferred_element_type=jnp.float32)
    psu[...] = jnp.dot(hsu[...], P[...], preferred_element_type=jnp.float32)
    psi[...] = jnp.dot(hsi[...], P[...], preferred_element_type=jnp.float32)
    pdu[...] = jnp.dot(hdu[...], P[...], preferred_element_type=jnp.float32)
    pdi[...] = jnp.dot(hdi[...], P[...], preferred_element_type=jnp.float32)


def _projections(hsu, hsi, hdu, hdi, fc_W, P):
    RB = 1000
    grid = N // RB
    feat_spec = pl.BlockSpec((RB, D_IN), lambda i: (i, 0))
    out8_spec = pl.BlockSpec((RB, 8), lambda i: (i, 0))
    return pl.pallas_call(
        _proj_body,
        grid=(grid,),
        in_specs=[feat_spec, feat_spec, feat_spec, feat_spec,
                  pl.BlockSpec((D_IN, H * D), lambda i: (0, 0)),
                  pl.BlockSpec((D_IN, 8), lambda i: (0, 0))],
        out_specs=[pl.BlockSpec((RB, H * D), lambda i: (i, 0)),
                   pl.BlockSpec((RB, H * D), lambda i: (i, 0)),
                   out8_spec, out8_spec, out8_spec, out8_spec],
        out_shape=[jax.ShapeDtypeStruct((N, H * D), jnp.float32),
                   jax.ShapeDtypeStruct((N, H * D), jnp.float32)]
                  + [jax.ShapeDtypeStruct((N, 8), jnp.float32)] * 4,
    )(hsu, hsi, hdu, hdi, fc_W, P)


def _sc_body(zu, zi, el_u, el_i, er_u, er_i, src_u2i, dst_u2i, src_i2u,
             dst_i2u, zrows0, zden0,
             num_i, num_u, den_i, den_u,
             el_t, er_t,
             z0, z1, sx0, sx1, si0, si1, dc0, dc1, dc2, dc3,
             ec0, ec1, ec2, ec3,
             num_s, den_s,
             g0, g1, s0, s1, dsem):
    c = lax.axis_index("c")
    s = lax.axis_index("s")
    zbuf = (z0, z1)
    srcb = (sx0, sx1)
    sidxb = (si0, si1)
    dcb = (dc0, dc1, dc2, dc3)
    eeb = (ec0, ec1, ec2, ec3)
    gsem = (g0, g1)
    ssem = (s0, s1)
    neg = jnp.full((16,), -3.4e38, jnp.float32)

    def vmax_table(tab):
        def body(i, m):
            return jnp.maximum(m, tab[pl.ds(i * 16, 16)])
        m = lax.fori_loop(0, N // 16, body, neg)
        acc = m[0]
        for i in range(1, 16):
            acc = jnp.maximum(acc, m[i])
        return acc

    for rel in range(2):
        if rel == 0:
            z, el_x, er_x = zu, el_u, er_i
            e_src, e_dst = src_u2i, dst_u2i
            num_o, den_o = num_i, den_i
        else:
            z, el_x, er_x = zi, el_i, er_u
            e_src, e_dst = src_i2u, dst_i2u
            num_o, den_o = num_u, den_u

        def task(h_loc, carry):
            h = c * 2 + h_loc
            pltpu.sync_copy(el_x.at[h], el_t)
            pltpu.sync_copy(er_x.at[h], er_t)
            M = jnp.maximum(vmax_table(el_t) + vmax_table(er_t), 0.0)

            for k in range(5):
                pltpu.sync_copy(zrows0,
                                num_s.at[pl.ds(s * STRIPE + k * 128, 128)])
            pltpu.sync_copy(zden0, den_s.at[pl.ds(s * STRIPE, STRIPE)])
            plsc.subcore_barrier()

            def prep_issue(ci, u, wait_prev):
                zs, es = u % 2, u % 4
                base = s * EPT + ci * C
                pltpu.sync_copy(e_src.at[pl.ds(base, C)], srcb[zs])
                pltpu.sync_copy(e_dst.at[pl.ds(base, C)], dcb[es])
                for j in range(C // 16):
                    sv = srcb[zs][pl.ds(j * 16, 16)]
                    dv = dcb[es][pl.ds(j * 16, 16)]
                    x = (plsc.load_gather(el_t, [sv])
                         + plsc.load_gather(er_t, [dv]))
                    e = jnp.where(x > 0, x, 0.2 * x)
                    eeb[es][pl.ds(j * 16, 16)] = jnp.exp(e - M)
                    sidxb[zs][pl.ds(j * 16, 16)] = sv * 4 + h
                if wait_prev:
                    pltpu.make_async_copy(zbuf[zs], num_s.at[dcb[es]],
                                          ssem[zs]).wait()
                pltpu.async_copy(z.at[sidxb[zs]], zbuf[zs], gsem[zs])

            def process(ci, u, prep, wait_prev=True, wait_den=True):
                zs, es = u % 2, u % 4
                if prep:
                    prep_issue(ci + 1, u + 1, wait_prev)
                zb = zbuf[zs]
                pltpu.make_async_copy(z.at[sidxb[zs]], zb, gsem[zs]).wait()
                pltpu.async_copy(eeb[es], den_s.at[dcb[es]], dsem, add=True)
                if wait_den:
                    pltpu.make_async_copy(eeb[es], den_s.at[dcb[es]],
                                          dsem).wait()

                def rb(q, carry2):
                    for v in range(4):
                        r = q * 4 + v
                        spl = plsc.load_gather(
                            eeb[es], [jnp.full((16,), r, jnp.int32)])
                        for k in range(8):
                            zb[r, pl.ds(k * 16, 16)] = (
                                zb[r, pl.ds(k * 16, 16)] * spl)
                    return carry2
                lax.fori_loop(0, C // 4, rb, 0)
                pltpu.async_copy(zb, num_s.at[dcb[es]], ssem[zs], add=True)

            prep_issue(0, 0, False)
            process(0, 0, True, wait_prev=False, wait_den=False)
            process(1, 1, True)
            process(2, 2, True)
            process(3, 3, True)

            def loop_body(i, carry2):
                for u in range(4):
                    process(i * 4 + u, u, True)
                return carry2
            lax.fori_loop(1, 30, loop_body, 0)

            process(120, 0, True)
            process(121, 1, True)
            process(122, 2, True)
            process(123, 3, True)
            process(124, 0, False)
            pltpu.make_async_copy(zbuf[1], num_s.at[dcb[3]], ssem[1]).wait()
            pltpu.make_async_copy(zbuf[0], num_s.at[dcb[0]], ssem[0]).wait()
            pltpu.make_async_copy(eeb[0], den_s.at[dcb[0]], dsem).wait()
            plsc.subcore_barrier()

            pltpu.sync_copy(num_s.at[pl.ds(s * STRIPE, STRIPE)],
                            num_o.at[h, pl.ds(s * STRIPE, STRIPE)])
            pltpu.sync_copy(den_s.at[pl.ds(s * STRIPE, STRIPE)],
                            den_o.at[h, pl.ds(s * STRIPE, STRIPE)])
            plsc.subcore_barrier()
            return carry
        lax.fori_loop(0, 2, task, 0)


def _sc_aggregate(zu_flat, zi_flat, el_u, el_i, er_u, er_i, e_u2i, e_i2u):
    src_u2i, dst_u2i = e_u2i[0], e_u2i[1]
    src_i2u, dst_i2u = e_i2u[0], e_i2u[1]
    zrows0 = jnp.zeros((128, D), jnp.float32)
    zden0 = jnp.zeros((STRIPE,), jnp.float32)
    mesh = plsc.VectorSubcoreMesh(core_axis_name="c", subcore_axis_name="s")
    f = pl.kernel(
        _sc_body,
        out_type=[jax.ShapeDtypeStruct((H, NP, D), jnp.float32),
                  jax.ShapeDtypeStruct((H, NP, D), jnp.float32),
                  jax.ShapeDtypeStruct((H, NP), jnp.float32),
                  jax.ShapeDtypeStruct((H, NP), jnp.float32)],
        mesh=mesh,
        compiler_params=pltpu.CompilerParams(needs_layout_passes=False),
        scratch_types=(
            [pltpu.VMEM((N,), jnp.float32),
             pltpu.VMEM((N,), jnp.float32)]
            + [pltpu.VMEM((C, D), jnp.float32)] * 2
            + [pltpu.VMEM((C,), jnp.int32)] * 2
            + [pltpu.VMEM((C,), jnp.int32)] * 2
            + [pltpu.VMEM((C,), jnp.int32)] * 4
            + [pltpu.VMEM((C,), jnp.float32)] * 4
            + [pltpu.VMEM_SHARED((NP, D), jnp.float32),
               pltpu.VMEM_SHARED((NP,), jnp.float32)]
            + [pltpu.SemaphoreType.DMA] * 5
        ),
    )
    return f(zu_flat, zi_flat, el_u, el_i, er_u, er_i,
             src_u2i, dst_u2i, src_i2u, dst_i2u, zrows0, zden0)


def _fin_body(num_u, den_u, hdu, num_i, den_i, hdi, W1, W2, b, zu, zi):
    def one(num_ref, den_ref, feat_ref, out_ref):
        den = jnp.maximum(den_ref[...], 1e-9)
        acc = num_ref[0] / den[:, 0:1]
        for h in range(1, H):
            acc = acc + num_ref[h] / den[:, h:h + 1]
        nu = acc * (1.0 / H)
        y = (jnp.dot(nu, W1[...], preferred_element_type=jnp.float32)
             + jnp.dot(feat_ref[...], W2[...], preferred_element_type=jnp.float32)
             + b[...])
        y = jnp.maximum(y, 0.0)
        nrm = jnp.sqrt(jnp.sum(y * y, axis=1, keepdims=True))
        out_ref[...] = y / jnp.where(nrm == 0.0, 1.0, nrm)
    one(num_u, den_u, hdu, zu)
    one(num_i, den_i, hdi, zi)


def _finalize(num_u, den_uT, hdu, num_i, den_iT, hdi, W1, W2, b):
    RB = 1000
    grid = N // RB
    num_spec = pl.BlockSpec((H, RB, D), lambda i: (0, i, 0))
    den_spec = pl.BlockSpec((RB, H), lambda i: (i, 0))
    feat_spec = pl.BlockSpec((RB, D_IN), lambda i: (i, 0))
    w_spec = pl.BlockSpec((128, D), lambda i: (0, 0))
    return pl.pallas_call(
        _fin_body,
        grid=(grid,),
        in_specs=[num_spec, den_spec, feat_spec,
                  num_spec, den_spec, feat_spec,
                  w_spec, w_spec, pl.BlockSpec((1, D), lambda i: (0, 0))],
        out_specs=[pl.BlockSpec((RB, D), lambda i: (i, 0)),
                   pl.BlockSpec((RB, D), lambda i: (i, 0))],
        out_shape=[jax.ShapeDtypeStruct((N, D), jnp.float32),
                   jax.ShapeDtypeStruct((N, D), jnp.float32)],
    )(num_u, den_uT, hdu, num_i, den_iT, hdi, W1, W2, b)


def kernel(h_src_user, h_src_item, h_dst_user, h_dst_item, edge_u2i, edge_i2u,
           fc_W, attn_l, attn_r, W_w, W_b):
    fc_r = fc_W.reshape(D_IN, H, D)
    Wl = jnp.einsum('khd,hd->kh', fc_r, attn_l)
    Wr = jnp.einsum('khd,hd->kh', fc_r, attn_r)
    P = jnp.concatenate([Wl, Wr], axis=1)

    zu, zi, psu, psi, pdu, pdi = _projections(
        h_src_user, h_src_item, h_dst_user, h_dst_item, fc_W, P)

    el_u = psu[:, 0:4].T
    el_i = psi[:, 0:4].T
    er_u = pdu[:, 4:8].T
    er_i = pdi[:, 4:8].T

    num_i, num_u, den_i, den_u = _sc_aggregate(
        zu.reshape(N * H, D), zi.reshape(N * H, D),
        el_u, el_i, er_u, er_i, edge_u2i, edge_i2u)

    W1 = W_w[:D, :]
    W2 = W_w[D:, :]
    z_user, z_item = _finalize(
        num_u, den_u[:, :N].T, h_dst_user,
        num_i, den_i[:, :N].T, h_dst_item,
        W1, W2, W_b.reshape(1, D))
    return (z_user, z_item)


# speedup vs baseline: 43.9605x; 1.2816x over previous
import functools

import jax
import jax.numpy as jnp
from jax import lax
from jax.experimental import pallas as pl
from jax.experimental.pallas import tpu as pltpu
from jax.experimental.pallas import tpu_sc as plsc

N = 10000
E = 160000
D_IN = 128
H = 4
D = 128
NP = 10240

NC = 2
NS = 16
EPT = E // NS
C = 80
NCHUNK = EPT // C
STRIPE = NP // NS


def _proj_body(hsu, hsi, hdu, hdi, fw, P, zu, zi, psu, psi, pdu, pdi):
    zu[...] = jnp.dot(hsu[...], fw[...], preferred_element_type=jnp.float32)
    zi[...] = jnp.dot(hsi[...], fw[...], preferred_element_type=jnp.float32)
    psu[...] = jnp.dot(hsu[...], P[...], preferred_element_type=jnp.float32)
    psi[...] = jnp.dot(hsi[...], P[...], preferred_element_type=jnp.float32)
    pdu[...] = jnp.dot(hdu[...], P[...], preferred_element_type=jnp.float32)
    pdi[...] = jnp.dot(hdi[...], P[...], preferred_element_type=jnp.float32)


def _projections(hsu, hsi, hdu, hdi, fc_W, P):
    RB = 1000
    grid = N // RB
    feat_spec = pl.BlockSpec((RB, D_IN), lambda i: (i, 0))
    out8_spec = pl.BlockSpec((RB, 8), lambda i: (i, 0))
    return pl.pallas_call(
        _proj_body,
        grid=(grid,),
        in_specs=[feat_spec, feat_spec, feat_spec, feat_spec,
                  pl.BlockSpec((D_IN, H * D), lambda i: (0, 0)),
                  pl.BlockSpec((D_IN, 8), lambda i: (0, 0))],
        out_specs=[pl.BlockSpec((RB, H * D), lambda i: (i, 0)),
                   pl.BlockSpec((RB, H * D), lambda i: (i, 0)),
                   out8_spec, out8_spec, out8_spec, out8_spec],
        out_shape=[jax.ShapeDtypeStruct((N, H * D), jnp.float32),
                   jax.ShapeDtypeStruct((N, H * D), jnp.float32)]
                  + [jax.ShapeDtypeStruct((N, 8), jnp.float32)] * 4,
    )(hsu, hsi, hdu, hdi, fc_W, P)


def _sc_body(zu, zi, el_u, el_i, er_u, er_i, src_u2i, dst_u2i, src_i2u,
             dst_i2u, zrows0, zden0,
             num_i, num_u, den_i, den_u,
             el_t, er_t,
             z0, z1, sx0, sx1, si0, si1, dc0, dc1, dc2, dc3,
             ec0, ec1, ec2, ec3,
             num_s, den_s,
             g0, g1, s0, s1, dsem, esem):
    c = lax.axis_index("c")
    s = lax.axis_index("s")
    zbuf = (z0, z1)
    srcb = (sx0, sx1)
    sidxb = (si0, si1)
    dcb = (dc0, dc1, dc2, dc3)
    eeb = (ec0, ec1, ec2, ec3)
    gsem = (g0, g1)
    ssem = (s0, s1)
    neg = jnp.full((16,), -3.4e38, jnp.float32)

    def vmax_table(tab):
        def body(i, m):
            return jnp.maximum(m, tab[pl.ds(i * 16, 16)])
        m = lax.fori_loop(0, N // 16, body, neg)
        acc = m[0]
        for i in range(1, 16):
            acc = jnp.maximum(acc, m[i])
        return acc

    for rel in range(2):
        if rel == 0:
            z, el_x, er_x = zu, el_u, er_i
            e_src, e_dst = src_u2i, dst_u2i
            num_o, den_o = num_i, den_i
        else:
            z, el_x, er_x = zi, el_i, er_u
            e_src, e_dst = src_i2u, dst_i2u
            num_o, den_o = num_u, den_u

        def task(h_loc, carry):
            h = c * 2 + h_loc
            pltpu.sync_copy(el_x.at[h], el_t)
            pltpu.sync_copy(er_x.at[h], er_t)
            M = jnp.maximum(vmax_table(el_t) + vmax_table(er_t), 0.0)

            for k in range(5):
                pltpu.sync_copy(zrows0,
                                num_s.at[pl.ds(s * STRIPE + k * 128, 128)])
            pltpu.sync_copy(zden0, den_s.at[pl.ds(s * STRIPE, STRIPE)])
            plsc.subcore_barrier()

            def issue_edges(ci, u):
                zs, es = u % 2, u % 4
                base = s * EPT + ci * C
                pltpu.async_copy(e_src.at[pl.ds(base, C)], srcb[zs], esem)
                pltpu.async_copy(e_dst.at[pl.ds(base, C)], dcb[es], esem)

            def wait_edges(u):
                zs, es = u % 2, u % 4
                base = s * EPT
                pltpu.make_async_copy(e_src.at[pl.ds(base, C)], srcb[zs],
                                      esem).wait()
                pltpu.make_async_copy(e_dst.at[pl.ds(base, C)], dcb[es],
                                      esem).wait()

            def compute_ee(u):
                zs, es = u % 2, u % 4
                for j in range(C // 16):
                    sv = srcb[zs][pl.ds(j * 16, 16)]
                    dv = dcb[es][pl.ds(j * 16, 16)]
                    x = (plsc.load_gather(el_t, [sv])
                         + plsc.load_gather(er_t, [dv]))
                    e = jnp.where(x > 0, x, 0.2 * x)
                    eeb[es][pl.ds(j * 16, 16)] = jnp.exp(e - M)
                    sidxb[zs][pl.ds(j * 16, 16)] = sv * 4 + h

            def process(ci, u, first=False):
                zs, es = u % 2, u % 4
                zs1, es1 = (u + 1) % 2, (u + 1) % 4

                def prep():
                    wait_edges(u + 1)
                    compute_ee(u + 1)
                    if not first:
                        pltpu.make_async_copy(eeb[es1], den_s.at[dcb[es1]],
                                              dsem).wait()
                        pltpu.make_async_copy(zbuf[zs1], num_s.at[dcb[es1]],
                                              ssem[zs1]).wait()
                    pltpu.async_copy(z.at[sidxb[zs1]], zbuf[zs1], gsem[zs1])

                    @pl.when(ci <= 122)
                    def _():
                        issue_edges(ci + 2, u + 2)

                if first:
                    prep()
                else:
                    @pl.when(ci <= 123)
                    def _():
                        prep()

                zb = zbuf[zs]
                pltpu.make_async_copy(z.at[sidxb[zs]], zb, gsem[zs]).wait()
                pltpu.async_copy(eeb[es], den_s.at[dcb[es]], dsem, add=True)

                def rb(q, carry2):
                    for v in range(8):
                        r = q * 8 + v
                        spl = plsc.load_gather(
                            eeb[es], [jnp.full((16,), r, jnp.int32)])
                        for k in range(8):
                            zb[r, pl.ds(k * 16, 16)] = (
                                zb[r, pl.ds(k * 16, 16)] * spl)
                    return carry2
                lax.fori_loop(0, C // 8, rb, 0)
                pltpu.async_copy(zb, num_s.at[dcb[es]], ssem[zs], add=True)

            pltpu.sync_copy(e_src.at[pl.ds(s * EPT, C)], srcb[0])
            pltpu.sync_copy(e_dst.at[pl.ds(s * EPT, C)], dcb[0])
            compute_ee(0)
            pltpu.async_copy(z.at[sidxb[0]], zbuf[0], gsem[0])
            issue_edges(1, 1)
            process(0, 0, first=True)

            def loop_body(i, carry2):
                for v in range(4):
                    process(i * 4 + 1 + v, 1 + v)
                return carry2
            lax.fori_loop(0, 31, loop_body, 0)

            pltpu.make_async_copy(zbuf[1], num_s.at[dcb[3]], ssem[1]).wait()
            pltpu.make_async_copy(zbuf[0], num_s.at[dcb[0]], ssem[0]).wait()
            pltpu.make_async_copy(eeb[3], den_s.at[dcb[3]], dsem).wait()
            pltpu.make_async_copy(eeb[0], den_s.at[dcb[0]], dsem).wait()
            plsc.subcore_barrier()

            pltpu.sync_copy(num_s.at[pl.ds(s * STRIPE, STRIPE)],
                            num_o.at[h, pl.ds(s * STRIPE, STRIPE)])
            pltpu.sync_copy(den_s.at[pl.ds(s * STRIPE, STRIPE)],
                            den_o.at[h, pl.ds(s * STRIPE, STRIPE)])
            plsc.subcore_barrier()
            return carry
        lax.fori_loop(0, 2, task, 0)


def _sc_aggregate(zu_flat, zi_flat, el_u, el_i, er_u, er_i, e_u2i, e_i2u):
    src_u2i, dst_u2i = e_u2i[0], e_u2i[1]
    src_i2u, dst_i2u = e_i2u[0], e_i2u[1]
    zrows0 = jnp.zeros((128, D), jnp.float32)
    zden0 = jnp.zeros((STRIPE,), jnp.float32)
    mesh = plsc.VectorSubcoreMesh(core_axis_name="c", subcore_axis_name="s")
    f = pl.kernel(
        _sc_body,
        out_type=[jax.ShapeDtypeStruct((H, NP, D), jnp.float32),
                  jax.ShapeDtypeStruct((H, NP, D), jnp.float32),
                  jax.ShapeDtypeStruct((H, NP), jnp.float32),
                  jax.ShapeDtypeStruct((H, NP), jnp.float32)],
        mesh=mesh,
        compiler_params=pltpu.CompilerParams(needs_layout_passes=False),
        scratch_types=(
            [pltpu.VMEM((N,), jnp.float32),
             pltpu.VMEM((N,), jnp.float32)]
            + [pltpu.VMEM((C, D), jnp.float32)] * 2
            + [pltpu.VMEM((C,), jnp.int32)] * 2
            + [pltpu.VMEM((C,), jnp.int32)] * 2
            + [pltpu.VMEM((C,), jnp.int32)] * 4
            + [pltpu.VMEM((C,), jnp.float32)] * 4
            + [pltpu.VMEM_SHARED((NP, D), jnp.float32),
               pltpu.VMEM_SHARED((NP,), jnp.float32)]
            + [pltpu.SemaphoreType.DMA] * 6
        ),
    )
    return f(zu_flat, zi_flat, el_u, el_i, er_u, er_i,
             src_u2i, dst_u2i, src_i2u, dst_i2u, zrows0, zden0)


def _fin_body(num_u, den_u, hdu, num_i, den_i, hdi, W1, W2, b, zu, zi):
    def one(num_ref, den_ref, feat_ref, out_ref):
        den = jnp.maximum(den_ref[...], 1e-9)
        acc = num_ref[0] / den[:, 0:1]
        for h in range(1, H):
            acc = acc + num_ref[h] / den[:, h:h + 1]
        nu = acc * (1.0 / H)
        y = (jnp.dot(nu, W1[...], preferred_element_type=jnp.float32)
             + jnp.dot(feat_ref[...], W2[...], preferred_element_type=jnp.float32)
             + b[...])
        y = jnp.maximum(y, 0.0)
        nrm = jnp.sqrt(jnp.sum(y * y, axis=1, keepdims=True))
        out_ref[...] = y / jnp.where(nrm == 0.0, 1.0, nrm)
    one(num_u, den_u, hdu, zu)
    one(num_i, den_i, hdi, zi)


def _finalize(num_u, den_uT, hdu, num_i, den_iT, hdi, W1, W2, b):
    RB = 1000
    grid = N // RB
    num_spec = pl.BlockSpec((H, RB, D), lambda i: (0, i, 0))
    den_spec = pl.BlockSpec((RB, H), lambda i: (i, 0))
    feat_spec = pl.BlockSpec((RB, D_IN), lambda i: (i, 0))
    w_spec = pl.BlockSpec((128, D), lambda i: (0, 0))
    return pl.pallas_call(
        _fin_body,
        grid=(grid,),
        in_specs=[num_spec, den_spec, feat_spec,
                  num_spec, den_spec, feat_spec,
                  w_spec, w_spec, pl.BlockSpec((1, D), lambda i: (0, 0))],
        out_specs=[pl.BlockSpec((RB, D), lambda i: (i, 0)),
                   pl.BlockSpec((RB, D), lambda i: (i, 0))],
        out_shape=[jax.ShapeDtypeStruct((N, D), jnp.float32),
                   jax.ShapeDtypeStruct((N, D), jnp.float32)],
    )(num_u, den_uT, hdu, num_i, den_iT, hdi, W1, W2, b)


def kernel(h_src_user, h_src_item, h_dst_user, h_dst_item, edge_u2i, edge_i2u,
           fc_W, attn_l, attn_r, W_w, W_b):
    fc_r = fc_W.reshape(D_IN, H, D)
    Wl = jnp.einsum('khd,hd->kh', fc_r, attn_l)
    Wr = jnp.einsum('khd,hd->kh', fc_r, attn_r)
    P = jnp.concatenate([Wl, Wr], axis=1)

    zu, zi, psu, psi, pdu, pdi = _projections(
        h_src_user, h_src_item, h_dst_user, h_dst_item, fc_W, P)

    el_u = psu[:, 0:4].T
    el_i = psi[:, 0:4].T
    er_u = pdu[:, 4:8].T
    er_i = pdi[:, 4:8].T

    num_i, num_u, den_i, den_u = _sc_aggregate(
        zu.reshape(N * H, D), zi.reshape(N * H, D),
        el_u, el_i, er_u, er_i, edge_u2i, edge_i2u)

    W1 = W_w[:D, :]
    W2 = W_w[D:, :]
    z_user, z_item = _finalize(
        num_u, den_u[:, :N].T, h_dst_user,
        num_i, den_i[:, :N].T, h_dst_item,
        W1, W2, W_b.reshape(1, D))
    return (z_user, z_item)


# speedup vs baseline: 48.2740x; 1.0981x over previous
import functools

import jax
import jax.numpy as jnp
from jax import lax
from jax.experimental import pallas as pl
from jax.experimental.pallas import tpu as pltpu
from jax.experimental.pallas import tpu_sc as plsc

N = 10000
E = 160000
D_IN = 128
H = 4
D = 128
NP = 10240

NC = 2
NS = 16
EPT = E // NS
C = 80
NCHUNK = EPT // C
STRIPE = NP // NS


def _proj_body(hsu, hsi, hdu, hdi, fw, P, zu, zi, psu, psi, pdu, pdi,
               msu, msi, mdu, mdi):
    i = pl.program_id(0)
    zu[...] = jnp.dot(hsu[...], fw[...], preferred_element_type=jnp.float32)
    zi[...] = jnp.dot(hsi[...], fw[...], preferred_element_type=jnp.float32)
    for feat, p_out, m_out in ((hsu, psu, msu), (hsi, psi, msi),
                               (hdu, pdu, mdu), (hdi, pdi, mdi)):
        p = jnp.dot(feat[...], P[...], preferred_element_type=jnp.float32)
        p_out[...] = p
        cmax = jnp.max(p, axis=0, keepdims=True)

        @pl.when(i == 0)
        def _():
            m_out[...] = cmax

        @pl.when(i > 0)
        def _():
            m_out[...] = jnp.maximum(m_out[...], cmax)


def _projections(hsu, hsi, hdu, hdi, fc_W, P):
    RB = 1000
    grid = N // RB
    feat_spec = pl.BlockSpec((RB, D_IN), lambda i: (i, 0))
    out8_spec = pl.BlockSpec((RB, 8), lambda i: (i, 0))
    max_spec = pl.BlockSpec((1, 8), lambda i: (0, 0))
    return pl.pallas_call(
        _proj_body,
        grid=(grid,),
        in_specs=[feat_spec, feat_spec, feat_spec, feat_spec,
                  pl.BlockSpec((D_IN, H * D), lambda i: (0, 0)),
                  pl.BlockSpec((D_IN, 8), lambda i: (0, 0))],
        out_specs=[pl.BlockSpec((RB, H * D), lambda i: (i, 0)),
                   pl.BlockSpec((RB, H * D), lambda i: (i, 0)),
                   out8_spec, out8_spec, out8_spec, out8_spec,
                   max_spec, max_spec, max_spec, max_spec],
        out_shape=[jax.ShapeDtypeStruct((N, H * D), jnp.float32),
                   jax.ShapeDtypeStruct((N, H * D), jnp.float32)]
                  + [jax.ShapeDtypeStruct((N, 8), jnp.float32)] * 4
                  + [jax.ShapeDtypeStruct((1, 8), jnp.float32)] * 4,
    )(hsu, hsi, hdu, hdi, fc_W, P)


def _sc_body(zu, zi, elf_u, elf_i, erf_u, erf_i, M0, M1,
             src_u2i, dst_u2i, src_i2u, dst_i2u, zrows0, zden0,
             num_i, num_u, den_i, den_u,
             z0, z1, z2, z3,
             sx0, sx1, sx2, sx3, dl0, dl1, dl2, dl3,
             ds0, ds1, ds2, ds3, si0, si1, si2, si3,
             ec0, ec1, ec2, ec3,
             ev0, ev1, rv0, rv1, ei0, ei1, ri0, ri1, m16,
             num_s, den_s,
             g0, g1, g2, g3, s0, s1, s2, s3, dsem, esem, elsem):
    c = lax.axis_index("c")
    s = lax.axis_index("s")
    zbuf = (z0, z1, z2, z3)
    srcb = (sx0, sx1, sx2, sx3)
    dlb = (dl0, dl1, dl2, dl3)
    dsb = (ds0, ds1, ds2, ds3)
    sidxb = (si0, si1, si2, si3)
    eeb = (ec0, ec1, ec2, ec3)
    elvb = (ev0, ev1)
    ervb = (rv0, rv1)
    elixb = (ei0, ei1)
    erixb = (ri0, ri1)
    gsem = (g0, g1, g2, g3)
    ssem = (s0, s1, s2, s3)

    for rel in range(2):
        if rel == 0:
            z, elf, erf, M_x = zu, elf_u, erf_i, M0
            e_src, e_dst = src_u2i, dst_u2i
            num_o, den_o = num_i, den_i
        else:
            z, elf, erf, M_x = zi, elf_i, erf_u, M1
            e_src, e_dst = src_i2u, dst_i2u
            num_o, den_o = num_u, den_u

        def task(h_loc, carry):
            h = c * 2 + h_loc
            pltpu.sync_copy(M_x, m16)
            hvec = jnp.full((16,), h, jnp.int32)
            Mv = plsc.load_gather(m16, [hvec])

            for k in range(5):
                pltpu.sync_copy(zrows0,
                                num_s.at[pl.ds(s * STRIPE + k * 128, 128)])
            pltpu.sync_copy(zden0, den_s.at[pl.ds(s * STRIPE, STRIPE)])
            plsc.subcore_barrier()

            def issue_edges(ci, u):
                e4 = u % 4
                base = s * EPT + ci * C
                pltpu.async_copy(e_src.at[pl.ds(base, C)], srcb[e4], esem)
                pltpu.async_copy(e_dst.at[pl.ds(base, C)], dlb[e4], esem)

            def wait_edges(u):
                e4 = u % 4
                base = s * EPT
                pltpu.make_async_copy(e_src.at[pl.ds(base, C)], srcb[e4],
                                      esem).wait()
                pltpu.make_async_copy(e_dst.at[pl.ds(base, C)], dlb[e4],
                                      esem).wait()

            def issue_elr(u):
                p2, e4 = u % 2, u % 4
                hN = h * N
                for j in range(C // 16):
                    sl = pl.ds(j * 16, 16)
                    elixb[p2][sl] = srcb[e4][sl] + hN
                    erixb[p2][sl] = dlb[e4][sl] + hN
                pltpu.async_copy(elf.at[elixb[p2]], elvb[p2], elsem)
                pltpu.async_copy(erf.at[erixb[p2]], ervb[p2], elsem)

            def wait_elr(u):
                p2 = u % 2
                pltpu.make_async_copy(elf.at[elixb[p2]], elvb[p2],
                                      elsem).wait()
                pltpu.make_async_copy(erf.at[erixb[p2]], ervb[p2],
                                      elsem).wait()

            def compute_ee(u):
                p2, e4 = u % 2, u % 4
                for j in range(C // 16):
                    sl = pl.ds(j * 16, 16)
                    x = elvb[p2][sl] + ervb[p2][sl]
                    e = jnp.where(x > 0, x, 0.2 * x)
                    eeb[e4][sl] = jnp.exp(e - Mv)
                    sidxb[e4][sl] = srcb[e4][sl] * 4 + h

            def process(ci, u, skip_c=False, skip_d=False, in_loop=False,
                        last=False):
                e4 = u % 4
                e41 = (u + 1) % 4

                if not last:
                    wait_elr(u + 1)
                    compute_ee(u + 1)
                    if not skip_c:
                        pltpu.make_async_copy(eeb[e41], den_s.at[dsb[e41]],
                                              dsem).wait()
                    if not skip_d:
                        pltpu.make_async_copy(zbuf[e41], num_s.at[dsb[e41]],
                                              ssem[e41]).wait()
                    for j in range(C // 16):
                        sl = pl.ds(j * 16, 16)
                        dsb[e41][sl] = dlb[e41][sl]
                    pltpu.async_copy(z.at[sidxb[e41]], zbuf[e41], gsem[e41])
                    if in_loop or ci <= 122:
                        wait_edges(u + 2)
                        issue_elr(u + 2)
                    if in_loop:
                        @pl.when(ci <= 121)
                        def _():
                            issue_edges(ci + 3, u + 3)
                    elif ci <= 121:
                        issue_edges(ci + 3, u + 3)

                zb = zbuf[e4]
                pltpu.make_async_copy(z.at[sidxb[e4]], zb, gsem[e4]).wait()
                pltpu.async_copy(eeb[e4], den_s.at[dsb[e4]], dsem, add=True)

                def rb(q, carry2):
                    for v in range(8):
                        r = q * 8 + v
                        spl = plsc.load_gather(
                            eeb[e4], [jnp.full((16,), r, jnp.int32)])
                        for k in range(8):
                            zb[r, pl.ds(k * 16, 16)] = (
                                zb[r, pl.ds(k * 16, 16)] * spl)
                    return carry2
                lax.fori_loop(0, C // 8, rb, 0)
                pltpu.async_copy(zb, num_s.at[dsb[e4]], ssem[e4], add=True)

            pltpu.sync_copy(e_src.at[pl.ds(s * EPT, C)], srcb[0])
            pltpu.sync_copy(e_dst.at[pl.ds(s * EPT, C)], dlb[0])
            issue_edges(1, 1)
            issue_edges(2, 2)
            issue_elr(0)
            wait_elr(0)
            compute_ee(0)
            for j in range(C // 16):
                sl = pl.ds(j * 16, 16)
                dsb[0][sl] = dlb[0][sl]
            pltpu.async_copy(z.at[sidxb[0]], zbuf[0], gsem[0])
            wait_edges(1)
            issue_elr(1)

            process(0, 0, skip_c=True, skip_d=True)
            process(1, 1, skip_d=True)
            process(2, 2, skip_d=True)

            def loop_body(i, carry2):
                for v in range(4):
                    process(i * 4 + 3 + v, 3 + v, in_loop=True)
                return carry2
            lax.fori_loop(0, 30, loop_body, 0)

            process(123, 3)
            process(124, 0, last=True)

            pltpu.make_async_copy(zbuf[1], num_s.at[dsb[1]], ssem[1]).wait()
            pltpu.make_async_copy(zbuf[2], num_s.at[dsb[2]], ssem[2]).wait()
            pltpu.make_async_copy(zbuf[3], num_s.at[dsb[3]], ssem[3]).wait()
            pltpu.make_async_copy(zbuf[0], num_s.at[dsb[0]], ssem[0]).wait()
            pltpu.make_async_copy(eeb[3], den_s.at[dsb[3]], dsem).wait()
            pltpu.make_async_copy(eeb[0], den_s.at[dsb[0]], dsem).wait()
            plsc.subcore_barrier()

            pltpu.sync_copy(num_s.at[pl.ds(s * STRIPE, STRIPE)],
                            num_o.at[h, pl.ds(s * STRIPE, STRIPE)])
            pltpu.sync_copy(den_s.at[pl.ds(s * STRIPE, STRIPE)],
                            den_o.at[h, pl.ds(s * STRIPE, STRIPE)])
            plsc.subcore_barrier()
            return carry
        lax.fori_loop(0, 2, task, 0)


def _sc_aggregate(zu_flat, zi_flat, elf_u, elf_i, erf_u, erf_i, M0, M1,
                  e_u2i, e_i2u):
    src_u2i, dst_u2i = e_u2i[0], e_u2i[1]
    src_i2u, dst_i2u = e_i2u[0], e_i2u[1]
    zrows0 = jnp.zeros((128, D), jnp.float32)
    zden0 = jnp.zeros((STRIPE,), jnp.float32)
    mesh = plsc.VectorSubcoreMesh(core_axis_name="c", subcore_axis_name="s")
    f = pl.kernel(
        _sc_body,
        out_type=[jax.ShapeDtypeStruct((H, NP, D), jnp.float32),
                  jax.ShapeDtypeStruct((H, NP, D), jnp.float32),
                  jax.ShapeDtypeStruct((H, NP), jnp.float32),
                  jax.ShapeDtypeStruct((H, NP), jnp.float32)],
        mesh=mesh,
        compiler_params=pltpu.CompilerParams(needs_layout_passes=False),
        scratch_types=(
            [pltpu.VMEM((C, D), jnp.float32)] * 4
            + [pltpu.VMEM((C,), jnp.int32)] * 4
            + [pltpu.VMEM((C,), jnp.int32)] * 4
            + [pltpu.VMEM((C,), jnp.int32)] * 4
            + [pltpu.VMEM((C,), jnp.int32)] * 4
            + [pltpu.VMEM((C,), jnp.float32)] * 4
            + [pltpu.VMEM((C,), jnp.float32)] * 2
            + [pltpu.VMEM((C,), jnp.float32)] * 2
            + [pltpu.VMEM((C,), jnp.int32)] * 2
            + [pltpu.VMEM((C,), jnp.int32)] * 2
            + [pltpu.VMEM((16,), jnp.float32)]
            + [pltpu.VMEM_SHARED((NP, D), jnp.float32),
               pltpu.VMEM_SHARED((NP,), jnp.float32)]
            + [pltpu.SemaphoreType.DMA] * 11
        ),
    )
    return f(zu_flat, zi_flat, elf_u, elf_i, erf_u, erf_i, M0, M1,
             src_u2i, dst_u2i, src_i2u, dst_i2u, zrows0, zden0)


def _fin_body(num_u, den_u, hdu, num_i, den_i, hdi, W1, W2, b, zu, zi):
    def one(num_ref, den_ref, feat_ref, out_ref):
        den = jnp.maximum(den_ref[...], 1e-9)
        acc = num_ref[0] / den[:, 0:1]
        for h in range(1, H):
            acc = acc + num_ref[h] / den[:, h:h + 1]
        nu = acc * (1.0 / H)
        y = (jnp.dot(nu, W1[...], preferred_element_type=jnp.float32)
             + jnp.dot(feat_ref[...], W2[...], preferred_element_type=jnp.float32)
             + b[...])
        y = jnp.maximum(y, 0.0)
        nrm = jnp.sqrt(jnp.sum(y * y, axis=1, keepdims=True))
        out_ref[...] = y / jnp.where(nrm == 0.0, 1.0, nrm)
    one(num_u, den_u, hdu, zu)
    one(num_i, den_i, hdi, zi)


def _finalize(num_u, den_uT, hdu, num_i, den_iT, hdi, W1, W2, b):
    RB = 1000
    grid = N // RB
    num_spec = pl.BlockSpec((H, RB, D), lambda i: (0, i, 0))
    den_spec = pl.BlockSpec((RB, H), lambda i: (i, 0))
    feat_spec = pl.BlockSpec((RB, D_IN), lambda i: (i, 0))
    w_spec = pl.BlockSpec((128, D), lambda i: (0, 0))
    return pl.pallas_call(
        _fin_body,
        grid=(grid,),
        in_specs=[num_spec, den_spec, feat_spec,
                  num_spec, den_spec, feat_spec,
                  w_spec, w_spec, pl.BlockSpec((1, D), lambda i: (0, 0))],
        out_specs=[pl.BlockSpec((RB, D), lambda i: (i, 0)),
                   pl.BlockSpec((RB, D), lambda i: (i, 0))],
        out_shape=[jax.ShapeDtypeStruct((N, D), jnp.float32),
                   jax.ShapeDtypeStruct((N, D), jnp.float32)],
    )(num_u, den_uT, hdu, num_i, den_iT, hdi, W1, W2, b)


def kernel(h_src_user, h_src_item, h_dst_user, h_dst_item, edge_u2i, edge_i2u,
           fc_W, attn_l, attn_r, W_w, W_b):
    fc_r = fc_W.reshape(D_IN, H, D)
    Wl = jnp.einsum('khd,hd->kh', fc_r, attn_l)
    Wr = jnp.einsum('khd,hd->kh', fc_r, attn_r)
    P = jnp.concatenate([Wl, Wr], axis=1)

    zu, zi, psu, psi, pdu, pdi, msu, msi, mdu, mdi = _projections(
        h_src_user, h_src_item, h_dst_user, h_dst_item, fc_W, P)

    elf_u = psu[:, 0:4].T.reshape(-1)
    elf_i = psi[:, 0:4].T.reshape(-1)
    erf_u = pdu[:, 4:8].T.reshape(-1)
    erf_i = pdi[:, 4:8].T.reshape(-1)
    M0 = jnp.pad(jnp.maximum(msu[0, 0:4] + mdi[0, 4:8], 0.0), (0, 12))
    M1 = jnp.pad(jnp.maximum(msi[0, 0:4] + mdu[0, 4:8], 0.0), (0, 12))

    num_i, num_u, den_i, den_u = _sc_aggregate(
        zu.reshape(N * H, D), zi.reshape(N * H, D),
        elf_u, elf_i, erf_u, erf_i, M0, M1, edge_u2i, edge_i2u)

    W1 = W_w[:D, :]
    W2 = W_w[D:, :]
    z_user, z_item = _finalize(
        num_u, den_u[:, :N].T, h_dst_user,
        num_i, den_i[:, :N].T, h_dst_item,
        W1, W2, W_b.reshape(1, D))
    return (z_user, z_item)


# speedup vs baseline: 48.4625x; 1.0039x over previous
import functools

import jax
import jax.numpy as jnp
from jax import lax
from jax.experimental import pallas as pl
from jax.experimental.pallas import tpu as pltpu
from jax.experimental.pallas import tpu_sc as plsc

N = 10000
E = 160000
D_IN = 128
H = 4
D = 128
NP = 10240

NC = 2
NS = 16
EPT = E // NS
C = 80
NCHUNK = EPT // C
STRIPE = NP // NS


def _proj_body(hsu, hsi, hdu, hdi, fw, P, zu, zi, psu, psi, pdu, pdi,
               msu, msi, mdu, mdi):
    i = pl.program_id(0)
    zu[...] = jnp.dot(hsu[...], fw[...], preferred_element_type=jnp.float32)
    zi[...] = jnp.dot(hsi[...], fw[...], preferred_element_type=jnp.float32)
    for feat, p_out, m_out in ((hsu, psu, msu), (hsi, psi, msi),
                               (hdu, pdu, mdu), (hdi, pdi, mdi)):
        p = jnp.dot(feat[...], P[...], preferred_element_type=jnp.float32)
        p_out[...] = p
        cmax = jnp.max(p, axis=0, keepdims=True)

        @pl.when(i == 0)
        def _():
            m_out[...] = cmax

        @pl.when(i > 0)
        def _():
            m_out[...] = jnp.maximum(m_out[...], cmax)


def _projections(hsu, hsi, hdu, hdi, fc_W, P):
    RB = 1000
    grid = N // RB
    feat_spec = pl.BlockSpec((RB, D_IN), lambda i: (i, 0))
    out8_spec = pl.BlockSpec((RB, 8), lambda i: (i, 0))
    max_spec = pl.BlockSpec((1, 8), lambda i: (0, 0))
    return pl.pallas_call(
        _proj_body,
        grid=(grid,),
        in_specs=[feat_spec, feat_spec, feat_spec, feat_spec,
                  pl.BlockSpec((D_IN, H * D), lambda i: (0, 0)),
                  pl.BlockSpec((D_IN, 8), lambda i: (0, 0))],
        out_specs=[pl.BlockSpec((RB, H * D), lambda i: (i, 0)),
                   pl.BlockSpec((RB, H * D), lambda i: (i, 0)),
                   out8_spec, out8_spec, out8_spec, out8_spec,
                   max_spec, max_spec, max_spec, max_spec],
        out_shape=[jax.ShapeDtypeStruct((N, H * D), jnp.float32),
                   jax.ShapeDtypeStruct((N, H * D), jnp.float32)]
                  + [jax.ShapeDtypeStruct((N, 8), jnp.float32)] * 4
                  + [jax.ShapeDtypeStruct((1, 8), jnp.float32)] * 4,
    )(hsu, hsi, hdu, hdi, fc_W, P)


def _sc_body(zu, zi, elf_u, elf_i, erf_u, erf_i, M0, M1,
             src_u2i, dst_u2i, src_i2u, dst_i2u, zrows0, zden0,
             num_i, num_u, den_i, den_u,
             z0, z1, z2, z3,
             sx0, sx1, sx2, sx3, dl0, dl1, dl2, dl3,
             ds0, ds1, ds2, ds3, si0, si1, si2, si3,
             ec0, ec1, ec2, ec3,
             ev0, ev1, rv0, rv1, ei0, ei1, ri0, ri1, m16,
             num_s, den_s,
             g0, g1, g2, g3, s0, s1, s2, s3, dsem, esem, elsem):
    c = lax.axis_index("c")
    s = lax.axis_index("s")
    zbuf = (z0, z1, z2, z3)
    srcb = (sx0, sx1, sx2, sx3)
    dlb = (dl0, dl1, dl2, dl3)
    dsb = (ds0, ds1, ds2, ds3)
    sidxb = (si0, si1, si2, si3)
    eeb = (ec0, ec1, ec2, ec3)
    elvb = (ev0, ev1)
    ervb = (rv0, rv1)
    elixb = (ei0, ei1)
    erixb = (ri0, ri1)
    gsem = (g0, g1, g2, g3)
    ssem = (s0, s1, s2, s3)

    for rel in range(2):
        if rel == 0:
            z, elf, erf, M_x = zu, elf_u, erf_i, M0
            e_src, e_dst = src_u2i, dst_u2i
            num_o, den_o = num_i, den_i
        else:
            z, elf, erf, M_x = zi, elf_i, erf_u, M1
            e_src, e_dst = src_i2u, dst_i2u
            num_o, den_o = num_u, den_u

        def task(h_loc, carry):
            h = c * 2 + h_loc
            pltpu.sync_copy(M_x, m16)
            hvec = jnp.full((16,), h, jnp.int32)
            Mv = plsc.load_gather(m16, [hvec])

            def issue_edges(ci, u):
                e4 = u % 4
                base = s * EPT + ci * C
                pltpu.async_copy(e_src.at[pl.ds(base, C)], srcb[e4], esem)
                pltpu.async_copy(e_dst.at[pl.ds(base, C)], dlb[e4], esem)

            def wait_edges(u):
                e4 = u % 4
                base = s * EPT
                pltpu.make_async_copy(e_src.at[pl.ds(base, C)], srcb[e4],
                                      esem).wait()
                pltpu.make_async_copy(e_dst.at[pl.ds(base, C)], dlb[e4],
                                      esem).wait()

            def issue_elr(u):
                p2, e4 = u % 2, u % 4
                hN = h * N
                for j in range(C // 16):
                    sl = pl.ds(j * 16, 16)
                    elixb[p2][sl] = srcb[e4][sl] + hN
                    erixb[p2][sl] = dlb[e4][sl] + hN
                pltpu.async_copy(elf.at[elixb[p2]], elvb[p2], elsem)
                pltpu.async_copy(erf.at[erixb[p2]], ervb[p2], elsem)

            def wait_elr(u):
                p2 = u % 2
                pltpu.make_async_copy(elf.at[elixb[p2]], elvb[p2],
                                      elsem).wait()
                pltpu.make_async_copy(erf.at[erixb[p2]], ervb[p2],
                                      elsem).wait()

            def compute_ee(u):
                p2, e4 = u % 2, u % 4
                for j in range(C // 16):
                    sl = pl.ds(j * 16, 16)
                    x = elvb[p2][sl] + ervb[p2][sl]
                    e = jnp.where(x > 0, x, 0.2 * x)
                    eeb[e4][sl] = jnp.exp(e - Mv)
                    sidxb[e4][sl] = srcb[e4][sl] * 4 + h

            def process(ci, u, skip_c=False, skip_d=False, in_loop=False,
                        last=False):
                e4 = u % 4
                e41 = (u + 1) % 4

                if not last:
                    wait_elr(u + 1)
                    compute_ee(u + 1)
                    if not skip_c:
                        pltpu.make_async_copy(eeb[e41], den_s.at[dsb[e41]],
                                              dsem).wait()
                    if not skip_d:
                        pltpu.make_async_copy(zbuf[e41], num_s.at[dsb[e41]],
                                              ssem[e41]).wait()
                    for j in range(C // 16):
                        sl = pl.ds(j * 16, 16)
                        dsb[e41][sl] = dlb[e41][sl]
                    pltpu.async_copy(z.at[sidxb[e41]], zbuf[e41], gsem[e41])
                    if in_loop or ci <= 122:
                        wait_edges(u + 2)
                        issue_elr(u + 2)
                    if in_loop:
                        @pl.when(ci <= 121)
                        def _():
                            issue_edges(ci + 3, u + 3)
                    elif ci <= 121:
                        issue_edges(ci + 3, u + 3)

                zb = zbuf[e4]
                pltpu.make_async_copy(z.at[sidxb[e4]], zb, gsem[e4]).wait()
                pltpu.async_copy(eeb[e4], den_s.at[dsb[e4]], dsem, add=True)

                def rb(q, carry2):
                    for v in range(8):
                        r = q * 8 + v
                        spl = plsc.load_gather(
                            eeb[e4], [jnp.full((16,), r, jnp.int32)])
                        for k in range(8):
                            zb[r, pl.ds(k * 16, 16)] = (
                                zb[r, pl.ds(k * 16, 16)] * spl)
                    return carry2
                lax.fori_loop(0, C // 8, rb, 0)
                pltpu.async_copy(zb, num_s.at[dsb[e4]], ssem[e4], add=True)

            pltpu.sync_copy(e_src.at[pl.ds(s * EPT, C)], srcb[0])
            pltpu.sync_copy(e_dst.at[pl.ds(s * EPT, C)], dlb[0])
            issue_edges(1, 1)
            issue_edges(2, 2)
            issue_elr(0)
            wait_elr(0)
            compute_ee(0)
            for j in range(C // 16):
                sl = pl.ds(j * 16, 16)
                dsb[0][sl] = dlb[0][sl]
            pltpu.async_copy(z.at[sidxb[0]], zbuf[0], gsem[0])
            wait_edges(1)
            issue_elr(1)

            for k in range(5):
                pltpu.sync_copy(zrows0,
                                num_s.at[pl.ds(s * STRIPE + k * 128, 128)])
            pltpu.sync_copy(zden0, den_s.at[pl.ds(s * STRIPE, STRIPE)])
            plsc.subcore_barrier()

            process(0, 0, skip_c=True, skip_d=True)
            process(1, 1, skip_d=True)
            process(2, 2, skip_d=True)

            def loop_body(i, carry2):
                for v in range(4):
                    process(i * 4 + 3 + v, 3 + v, in_loop=True)
                return carry2
            lax.fori_loop(0, 30, loop_body, 0)

            process(123, 3)
            process(124, 0, last=True)

            pltpu.make_async_copy(zbuf[1], num_s.at[dsb[1]], ssem[1]).wait()
            pltpu.make_async_copy(zbuf[2], num_s.at[dsb[2]], ssem[2]).wait()
            pltpu.make_async_copy(zbuf[3], num_s.at[dsb[3]], ssem[3]).wait()
            pltpu.make_async_copy(zbuf[0], num_s.at[dsb[0]], ssem[0]).wait()
            pltpu.make_async_copy(eeb[3], den_s.at[dsb[3]], dsem).wait()
            pltpu.make_async_copy(eeb[0], den_s.at[dsb[0]], dsem).wait()
            plsc.subcore_barrier()

            pltpu.sync_copy(num_s.at[pl.ds(s * STRIPE, STRIPE)],
                            num_o.at[h, pl.ds(s * STRIPE, STRIPE)])
            pltpu.sync_copy(den_s.at[pl.ds(s * STRIPE, STRIPE)],
                            den_o.at[h, pl.ds(s * STRIPE, STRIPE)])
            plsc.subcore_barrier()
            return carry
        lax.fori_loop(0, 2, task, 0)


def _sc_aggregate(zu_flat, zi_flat, elf_u, elf_i, erf_u, erf_i, M0, M1,
                  e_u2i, e_i2u):
    src_u2i, dst_u2i = e_u2i[0], e_u2i[1]
    src_i2u, dst_i2u = e_i2u[0], e_i2u[1]
    zrows0 = jnp.zeros((128, D), jnp.float32)
    zden0 = jnp.zeros((STRIPE,), jnp.float32)
    mesh = plsc.VectorSubcoreMesh(core_axis_name="c", subcore_axis_name="s")
    f = pl.kernel(
        _sc_body,
        out_type=[jax.ShapeDtypeStruct((H, NP, D), jnp.float32),
                  jax.ShapeDtypeStruct((H, NP, D), jnp.float32),
                  jax.ShapeDtypeStruct((H, NP), jnp.float32),
                  jax.ShapeDtypeStruct((H, NP), jnp.float32)],
        mesh=mesh,
        compiler_params=pltpu.CompilerParams(needs_layout_passes=False),
        scratch_types=(
            [pltpu.VMEM((C, D), jnp.float32)] * 4
            + [pltpu.VMEM((C,), jnp.int32)] * 4
            + [pltpu.VMEM((C,), jnp.int32)] * 4
            + [pltpu.VMEM((C,), jnp.int32)] * 4
            + [pltpu.VMEM((C,), jnp.int32)] * 4
            + [pltpu.VMEM((C,), jnp.float32)] * 4
            + [pltpu.VMEM((C,), jnp.float32)] * 2
            + [pltpu.VMEM((C,), jnp.float32)] * 2
            + [pltpu.VMEM((C,), jnp.int32)] * 2
            + [pltpu.VMEM((C,), jnp.int32)] * 2
            + [pltpu.VMEM((16,), jnp.float32)]
            + [pltpu.VMEM_SHARED((NP, D), jnp.float32),
               pltpu.VMEM_SHARED((NP,), jnp.float32)]
            + [pltpu.SemaphoreType.DMA] * 11
        ),
    )
    return f(zu_flat, zi_flat, elf_u, elf_i, erf_u, erf_i, M0, M1,
             src_u2i, dst_u2i, src_i2u, dst_i2u, zrows0, zden0)


def _fin_body(num_u, den_u, hdu, num_i, den_i, hdi, W1, W2, b, zu, zi):
    def one(num_ref, den_ref, feat_ref, out_ref):
        den = jnp.maximum(den_ref[...], 1e-9)
        acc = num_ref[0] / den[:, 0:1]
        for h in range(1, H):
            acc = acc + num_ref[h] / den[:, h:h + 1]
        nu = acc * (1.0 / H)
        y = (jnp.dot(nu, W1[...], preferred_element_type=jnp.float32)
             + jnp.dot(feat_ref[...], W2[...], preferred_element_type=jnp.float32)
             + b[...])
        y = jnp.maximum(y, 0.0)
        nrm = jnp.sqrt(jnp.sum(y * y, axis=1, keepdims=True))
        out_ref[...] = y / jnp.where(nrm == 0.0, 1.0, nrm)
    one(num_u, den_u, hdu, zu)
    one(num_i, den_i, hdi, zi)


def _finalize(num_u, den_uT, hdu, num_i, den_iT, hdi, W1, W2, b):
    RB = 1000
    grid = N // RB
    num_spec = pl.BlockSpec((H, RB, D), lambda i: (0, i, 0))
    den_spec = pl.BlockSpec((RB, H), lambda i: (i, 0))
    feat_spec = pl.BlockSpec((RB, D_IN), lambda i: (i, 0))
    w_spec = pl.BlockSpec((128, D), lambda i: (0, 0))
    return pl.pallas_call(
        _fin_body,
        grid=(grid,),
        in_specs=[num_spec, den_spec, feat_spec,
                  num_spec, den_spec, feat_spec,
                  w_spec, w_spec, pl.BlockSpec((1, D), lambda i: (0, 0))],
        out_specs=[pl.BlockSpec((RB, D), lambda i: (i, 0)),
                   pl.BlockSpec((RB, D), lambda i: (i, 0))],
        out_shape=[jax.ShapeDtypeStruct((N, D), jnp.float32),
                   jax.ShapeDtypeStruct((N, D), jnp.float32)],
    )(num_u, den_uT, hdu, num_i, den_iT, hdi, W1, W2, b)


def kernel(h_src_user, h_src_item, h_dst_user, h_dst_item, edge_u2i, edge_i2u,
           fc_W, attn_l, attn_r, W_w, W_b):
    fc_r = fc_W.reshape(D_IN, H, D)
    Wl = jnp.einsum('khd,hd->kh', fc_r, attn_l)
    Wr = jnp.einsum('khd,hd->kh', fc_r, attn_r)
    P = jnp.concatenate([Wl, Wr], axis=1)

    zu, zi, psu, psi, pdu, pdi, msu, msi, mdu, mdi = _projections(
        h_src_user, h_src_item, h_dst_user, h_dst_item, fc_W, P)

    elf_u = psu[:, 0:4].T.reshape(-1)
    elf_i = psi[:, 0:4].T.reshape(-1)
    erf_u = pdu[:, 4:8].T.reshape(-1)
    erf_i = pdi[:, 4:8].T.reshape(-1)
    M0 = jnp.pad(jnp.maximum(msu[0, 0:4] + mdi[0, 4:8], 0.0), (0, 12))
    M1 = jnp.pad(jnp.maximum(msi[0, 0:4] + mdu[0, 4:8], 0.0), (0, 12))

    num_i, num_u, den_i, den_u = _sc_aggregate(
        zu.reshape(N * H, D), zi.reshape(N * H, D),
        elf_u, elf_i, erf_u, erf_i, M0, M1, edge_u2i, edge_i2u)

    W1 = W_w[:D, :]
    W2 = W_w[D:, :]
    z_user, z_item = _finalize(
        num_u, den_u[:, :N].T, h_dst_user,
        num_i, den_i[:, :N].T, h_dst_item,
        W1, W2, W_b.reshape(1, D))
    return (z_user, z_item)
